# Initial kernel scaffold; baseline (speedup 1.0000x reference)
import jax
import jax.numpy as jnp
import numpy as np
from jax.experimental import pallas as pl
from jax.experimental.pallas import tpu as pltpu

_TEMPERATURE = 0.01
_B, _C, _H, _W, _NUM = 4, 256, 64, 64, 48
_HW = _H * _W
_KK = _NUM // 3
_NPLANE = 81
_PLANE_PAD = 88
_PADW = 4736


def _cost_kernel(l_ref, rp_ref, out_ref):
    lf = l_ref[0]

    def body(sidx, _):
        sh = (sidx // 9) * 64 + sidx % 9
        rs = rp_ref[0, :, pl.ds(sh, _HW)]
        row = jnp.sum(lf * rs, axis=0)
        out_ref[0, pl.ds(sidx, 1), :] = row.reshape(1, _HW)
        return 0

    jax.lax.fori_loop(0, _NPLANE, body, 0)


def _select_kernel(cost_ref, offx_ref, offy_ref, ox_ref, oy_ref, corr_ref,
                   code_ref, v_ref):
    p = jax.lax.broadcasted_iota(jnp.int32, (_NUM, _HW), 1)
    y = p // _W
    x = p % _W
    dye = jnp.minimum(offy_ref[0], (_H - 1) - y)
    dxe = jnp.minimum(offx_ref[0], (_W - 1) - x)
    code = dye * 9 + dxe
    code_ref[...] = code

    def sel_body(s, u):
        plane = cost_ref[0, pl.ds(s, 1), :]
        return jnp.where(code == s, plane, u)

    u = jax.lax.fori_loop(0, _NPLANE, sel_body,
                          jnp.zeros((_NUM, _HW), jnp.float32))
    u = u / np.float32(_TEMPERATURE)

    m = jnp.max(u, axis=1, keepdims=True)
    e = jnp.exp(u - m)
    s = jnp.sum(e, axis=1, keepdims=True)
    v_ref[...] = e / s

    i_iota = jax.lax.broadcasted_iota(jnp.int32, (_NUM, _HW), 0)
    v = v_ref[...]

    def rank_body(j, rank):
        vj = v_ref[pl.ds(j, 1), :]
        beats = (vj > v) | ((vj == v) & (j < i_iota))
        return rank + beats.astype(jnp.int32)

    rank = jax.lax.fori_loop(0, _NUM, rank_body,
                             jnp.zeros((_NUM, _HW), jnp.int32))

    codef = code_ref[...]

    def out_body(k, _):
        selm = rank == k
        corr_k = jnp.sum(jnp.where(selm, v, 0.0), axis=0, keepdims=True)
        code_k = jnp.sum(jnp.where(selm, codef, 0), axis=0, keepdims=True)
        ox_ref[0, pl.ds(k, 1), :] = (code_k % 9).astype(jnp.float32)
        oy_ref[0, pl.ds(k, 1), :] = (code_k // 9).astype(jnp.float32)
        corr_ref[0, pl.ds(k, 1), :] = corr_k
        return 0

    jax.lax.fori_loop(0, _KK, out_body, 0)


@jax.jit
def kernel(left_features, right_features, offset_x, offset_y):
    rb = right_features.reshape(_C, _B, _HW).transpose(1, 0, 2)
    rp = jnp.pad(rb, ((0, 0), (0, 0), (0, _PADW - _HW)))
    offx = offset_x.reshape(_B, _NUM, _HW)
    offy = offset_y.reshape(_B, _NUM, _HW)

    cost_all = pl.pallas_call(
        _cost_kernel,
        grid=(_B,),
        in_specs=[
            pl.BlockSpec((1, _C, _HW), lambda b: (b, 0, 0)),
            pl.BlockSpec((1, _C, _PADW), lambda b: (b, 0, 0)),
        ],
        out_specs=pl.BlockSpec((1, _PLANE_PAD, _HW), lambda b: (b, 0, 0)),
        out_shape=jax.ShapeDtypeStruct((_B, _PLANE_PAD, _HW), jnp.float32),
    )(left_features, rp)

    ox_o, oy_o, corr = pl.pallas_call(
        _select_kernel,
        grid=(_B,),
        in_specs=[
            pl.BlockSpec((1, _PLANE_PAD, _HW), lambda b: (b, 0, 0)),
            pl.BlockSpec((1, _NUM, _HW), lambda b: (b, 0, 0)),
            pl.BlockSpec((1, _NUM, _HW), lambda b: (b, 0, 0)),
        ],
        out_specs=[
            pl.BlockSpec((1, _KK, _HW), lambda b: (b, 0, 0)),
            pl.BlockSpec((1, _KK, _HW), lambda b: (b, 0, 0)),
            pl.BlockSpec((1, _KK, _HW), lambda b: (b, 0, 0)),
        ],
        out_shape=[
            jax.ShapeDtypeStruct((_B, _KK, _HW), jnp.float32),
            jax.ShapeDtypeStruct((_B, _KK, _HW), jnp.float32),
            jax.ShapeDtypeStruct((_B, _KK, _HW), jnp.float32),
        ],
        scratch_shapes=[
            pltpu.VMEM((_NUM, _HW), jnp.int32),
            pltpu.VMEM((_NUM, _HW), jnp.float32),
        ],
    )(cost_all, offx, offy)

    return (ox_o.reshape(_B, _KK, _H, _W), oy_o.reshape(_B, _KK, _H, _W),
            corr)


# speedup vs baseline: 1.8474x; 1.8474x over previous
import jax
import jax.numpy as jnp
import numpy as np
from jax.experimental import pallas as pl
from jax.experimental.pallas import tpu as pltpu

_TEMPERATURE = 0.01
_B, _C, _H, _W, _NUM = 4, 256, 64, 64, 48
_HW = _H * _W
_KK = _NUM // 3
_NPLANE = 81
_PLANE_PAD = 88
_PADW = 4736


def _cost_kernel(l_ref, rp_ref, out_ref):
    lf = l_ref[0]
    rpv = rp_ref[0]

    def body(sidx, _):
        sh = (sidx // 9) * 64 + sidx % 9
        rolled = pltpu.roll(rpv, (_PADW - sh) % _PADW, axis=1)
        row = jnp.sum(lf * rolled[:, :_HW], axis=0)
        out_ref[0, pl.ds(sidx, 1), :] = row.reshape(1, _HW)
        return 0

    jax.lax.fori_loop(0, _NPLANE, body, 0)


def _select_kernel(cost_ref, offx_ref, offy_ref, ox_ref, oy_ref, corr_ref,
                   code_ref, v_ref):
    p = jax.lax.broadcasted_iota(jnp.int32, (_NUM, _HW), 1)
    y = p // _W
    x = p % _W
    dye = jnp.minimum(offy_ref[0], (_H - 1) - y)
    dxe = jnp.minimum(offx_ref[0], (_W - 1) - x)
    code = dye * 9 + dxe
    code_ref[...] = code

    def sel_body(s, u):
        plane = cost_ref[0, pl.ds(s, 1), :]
        return jnp.where(code == s, plane, u)

    u = jax.lax.fori_loop(0, _NPLANE, sel_body,
                          jnp.zeros((_NUM, _HW), jnp.float32))
    u = u / np.float32(_TEMPERATURE)

    m = jnp.max(u, axis=1, keepdims=True)
    e = jnp.exp(u - m)
    s = jnp.sum(e, axis=1, keepdims=True)
    v_ref[...] = e / s

    i_iota = jax.lax.broadcasted_iota(jnp.int32, (_NUM, _HW), 0)
    v = v_ref[...]

    def rank_body(j, rank):
        vj = v_ref[pl.ds(j, 1), :]
        beats = (vj > v) | ((vj == v) & (j < i_iota))
        return rank + beats.astype(jnp.int32)

    rank = jax.lax.fori_loop(0, _NUM, rank_body,
                             jnp.zeros((_NUM, _HW), jnp.int32))

    codef = code_ref[...]

    def out_body(k, _):
        selm = rank == k
        corr_k = jnp.sum(jnp.where(selm, v, 0.0), axis=0, keepdims=True)
        code_k = jnp.sum(jnp.where(selm, codef, 0), axis=0, keepdims=True)
        ox_ref[0, pl.ds(k, 1), :] = (code_k % 9).astype(jnp.float32)
        oy_ref[0, pl.ds(k, 1), :] = (code_k // 9).astype(jnp.float32)
        corr_ref[0, pl.ds(k, 1), :] = corr_k
        return 0

    jax.lax.fori_loop(0, _KK, out_body, 0)


@jax.jit
def kernel(left_features, right_features, offset_x, offset_y):
    rb = right_features.reshape(_C, _B, _HW).transpose(1, 0, 2)
    rp = jnp.pad(rb, ((0, 0), (0, 0), (0, _PADW - _HW)))
    offx = offset_x.reshape(_B, _NUM, _HW)
    offy = offset_y.reshape(_B, _NUM, _HW)

    cost_all = pl.pallas_call(
        _cost_kernel,
        grid=(_B,),
        in_specs=[
            pl.BlockSpec((1, _C, _HW), lambda b: (b, 0, 0)),
            pl.BlockSpec((1, _C, _PADW), lambda b: (b, 0, 0)),
        ],
        out_specs=pl.BlockSpec((1, _PLANE_PAD, _HW), lambda b: (b, 0, 0)),
        out_shape=jax.ShapeDtypeStruct((_B, _PLANE_PAD, _HW), jnp.float32),
    )(left_features, rp)

    ox_o, oy_o, corr = pl.pallas_call(
        _select_kernel,
        grid=(_B,),
        in_specs=[
            pl.BlockSpec((1, _PLANE_PAD, _HW), lambda b: (b, 0, 0)),
            pl.BlockSpec((1, _NUM, _HW), lambda b: (b, 0, 0)),
            pl.BlockSpec((1, _NUM, _HW), lambda b: (b, 0, 0)),
        ],
        out_specs=[
            pl.BlockSpec((1, _KK, _HW), lambda b: (b, 0, 0)),
            pl.BlockSpec((1, _KK, _HW), lambda b: (b, 0, 0)),
            pl.BlockSpec((1, _KK, _HW), lambda b: (b, 0, 0)),
        ],
        out_shape=[
            jax.ShapeDtypeStruct((_B, _KK, _HW), jnp.float32),
            jax.ShapeDtypeStruct((_B, _KK, _HW), jnp.float32),
            jax.ShapeDtypeStruct((_B, _KK, _HW), jnp.float32),
        ],
        scratch_shapes=[
            pltpu.VMEM((_NUM, _HW), jnp.int32),
            pltpu.VMEM((_NUM, _HW), jnp.float32),
        ],
    )(cost_all, offx, offy)

    return (ox_o.reshape(_B, _KK, _H, _W), oy_o.reshape(_B, _KK, _H, _W),
            corr)


# speedup vs baseline: 3.9203x; 2.1221x over previous
import jax
import jax.numpy as jnp
import numpy as np
from jax.experimental import pallas as pl
from jax.experimental.pallas import tpu as pltpu

_TEMPERATURE = 0.01
_B, _C, _H, _W, _NUM = 4, 256, 64, 64, 48
_HW = _H * _W
_KK = _NUM // 3
_NPLANE = 81
_PLANE_PAD = 88
_PADW = 4736


def _cost_kernel(l_ref, rp_ref, out_ref, rdx_ref, rdx64_ref):
    dx = pl.program_id(1)

    @pl.when(dx == 0)
    def _():
        rdx_ref[...] = rp_ref[0]
        rdx64_ref[...] = pltpu.roll(rp_ref[0], _PADW - 64, axis=1)

    @pl.when(dx != 0)
    def _():
        rdx_ref[...] = pltpu.roll(rdx_ref[...], _PADW - 1, axis=1)
        rdx64_ref[...] = pltpu.roll(rdx64_ref[...], _PADW - 1, axis=1)

    lf = l_ref[0]

    def even_body(e, _):
        off = pl.multiple_of(e * 128, 128)
        row = jnp.sum(lf * rdx_ref[:, pl.ds(off, _HW)], axis=0)
        out_ref[0, pl.ds(2 * e * 9 + dx, 1), :] = row.reshape(1, _HW)
        return 0

    jax.lax.fori_loop(0, 5, even_body, 0)

    def odd_body(o, _):
        off = pl.multiple_of(o * 128, 128)
        row = jnp.sum(lf * rdx64_ref[:, pl.ds(off, _HW)], axis=0)
        out_ref[0, pl.ds((2 * o + 1) * 9 + dx, 1), :] = row.reshape(1, _HW)
        return 0

    jax.lax.fori_loop(0, 4, odd_body, 0)


def _select_kernel(cost_ref, offx_ref, offy_ref, ox_ref, oy_ref, corr_ref,
                   code_ref, v_ref):
    p = jax.lax.broadcasted_iota(jnp.int32, (_NUM, _HW), 1)
    y = p // _W
    x = p % _W
    dye = jnp.minimum(offy_ref[0], (_H - 1) - y)
    dxe = jnp.minimum(offx_ref[0], (_W - 1) - x)
    code = dye * 9 + dxe
    code_ref[...] = code

    def sel_body(s, u):
        plane = cost_ref[0, pl.ds(s, 1), :]
        return jnp.where(code == s, plane, u)

    u = jax.lax.fori_loop(0, _NPLANE, sel_body,
                          jnp.zeros((_NUM, _HW), jnp.float32))
    u = u / np.float32(_TEMPERATURE)

    m = jnp.max(u, axis=1, keepdims=True)
    e = jnp.exp(u - m)
    s = jnp.sum(e, axis=1, keepdims=True)
    v_ref[...] = e / s

    i_iota = jax.lax.broadcasted_iota(jnp.int32, (_NUM, _HW), 0)
    v = v_ref[...]

    def rank_body(j, rank):
        vj = v_ref[pl.ds(j, 1), :]
        beats = (vj > v) | ((vj == v) & (j < i_iota))
        return rank + beats.astype(jnp.int32)

    rank = jax.lax.fori_loop(0, _NUM, rank_body,
                             jnp.zeros((_NUM, _HW), jnp.int32))

    codef = code_ref[...]

    def out_body(k, _):
        selm = rank == k
        corr_k = jnp.sum(jnp.where(selm, v, 0.0), axis=0, keepdims=True)
        code_k = jnp.sum(jnp.where(selm, codef, 0), axis=0, keepdims=True)
        ox_ref[0, pl.ds(k, 1), :] = (code_k % 9).astype(jnp.float32)
        oy_ref[0, pl.ds(k, 1), :] = (code_k // 9).astype(jnp.float32)
        corr_ref[0, pl.ds(k, 1), :] = corr_k
        return 0

    jax.lax.fori_loop(0, _KK, out_body, 0)


@jax.jit
def kernel(left_features, right_features, offset_x, offset_y):
    rb = right_features.reshape(_C, _B, _HW).transpose(1, 0, 2)
    rp = jnp.pad(rb, ((0, 0), (0, 0), (0, _PADW - _HW)))
    offx = offset_x.reshape(_B, _NUM, _HW)
    offy = offset_y.reshape(_B, _NUM, _HW)

    cost_all = pl.pallas_call(
        _cost_kernel,
        grid=(_B, 9),
        in_specs=[
            pl.BlockSpec((1, _C, _HW), lambda b, dx: (b, 0, 0)),
            pl.BlockSpec((1, _C, _PADW), lambda b, dx: (b, 0, 0)),
        ],
        out_specs=pl.BlockSpec((1, _PLANE_PAD, _HW), lambda b, dx: (b, 0, 0)),
        out_shape=jax.ShapeDtypeStruct((_B, _PLANE_PAD, _HW), jnp.float32),
        scratch_shapes=[
            pltpu.VMEM((_C, _PADW), jnp.float32),
            pltpu.VMEM((_C, _PADW), jnp.float32),
        ],
    )(left_features, rp)

    ox_o, oy_o, corr = pl.pallas_call(
        _select_kernel,
        grid=(_B,),
        in_specs=[
            pl.BlockSpec((1, _PLANE_PAD, _HW), lambda b: (b, 0, 0)),
            pl.BlockSpec((1, _NUM, _HW), lambda b: (b, 0, 0)),
            pl.BlockSpec((1, _NUM, _HW), lambda b: (b, 0, 0)),
        ],
        out_specs=[
            pl.BlockSpec((1, _KK, _HW), lambda b: (b, 0, 0)),
            pl.BlockSpec((1, _KK, _HW), lambda b: (b, 0, 0)),
            pl.BlockSpec((1, _KK, _HW), lambda b: (b, 0, 0)),
        ],
        out_shape=[
            jax.ShapeDtypeStruct((_B, _KK, _HW), jnp.float32),
            jax.ShapeDtypeStruct((_B, _KK, _HW), jnp.float32),
            jax.ShapeDtypeStruct((_B, _KK, _HW), jnp.float32),
        ],
        scratch_shapes=[
            pltpu.VMEM((_NUM, _HW), jnp.int32),
            pltpu.VMEM((_NUM, _HW), jnp.float32),
        ],
    )(cost_all, offx, offy)

    return (ox_o.reshape(_B, _KK, _H, _W), oy_o.reshape(_B, _KK, _H, _W),
            corr)


# speedup vs baseline: 5.6630x; 1.4445x over previous
import jax
import jax.numpy as jnp
import numpy as np
from jax.experimental import pallas as pl
from jax.experimental.pallas import tpu as pltpu

_TEMPERATURE = 0.01
_B, _C, _H, _W, _NUM = 4, 256, 64, 64, 48
_HW = _H * _W
_KK = _NUM // 3
_NPLANE = 81
_PLANE_PAD = 88
_PADW = 4736


_LW = 4224
_NG = _LW // 128
_NCT = _C // 8


def _cost_kernel(l_ref, rf_ref, out_ref, lx_ref, rbase_ref, r64_ref,
                 rowbuf_ref):
    dx = pl.program_id(1)

    @pl.when(dx == 0)
    def _():
        rbase_ref[:, : _HW] = rf_ref[...]
        rbase_ref[:, _HW:] = jnp.zeros((_C, _PADW - _HW), jnp.float32)
        r64_ref[...] = pltpu.roll(rbase_ref[...], _PADW - 64, axis=1)
        lx_ref[:, : _HW] = l_ref[0]
        lx_ref[:, _HW:] = jnp.zeros((_C, _LW - _HW), jnp.float32)

    @pl.when(dx != 0)
    def _():
        lx_ref[...] = pltpu.roll(lx_ref[...], 1, axis=1)

    def g_body(g, _):
        goff = pl.multiple_of(g * 128, 128)

        accs = [[jnp.zeros((8, 128), jnp.float32) for _ in range(9)]
                for _ in range(2)]
        for ct in range(_NCT):
            lt = lx_ref[pl.ds(ct * 8, 8), pl.ds(goff, 128)]
            for dy in range(9):
                src = rbase_ref if dy % 2 == 0 else r64_ref
                off = pl.multiple_of((g + dy // 2) * 128, 128)
                rt = src[pl.ds(ct * 8, 8), pl.ds(off, 128)]
                accs[ct % 2][dy] = accs[ct % 2][dy] + lt * rt
        for dy in range(9):
            seg = jnp.sum(accs[0][dy] + accs[1][dy], axis=0,
                          keepdims=True)
            rowbuf_ref[pl.ds(dy, 1), pl.ds(goff, 128)] = seg
        return 0

    jax.lax.fori_loop(0, _NG, g_body, 0)

    amt = jax.lax.rem(_LW - dx, _LW)
    for dy in range(9):
        row = rowbuf_ref[pl.ds(dy, 1), :]
        rolled = pltpu.roll(row, amt, axis=1)
        out_ref[0, pl.ds(dy * 9 + dx, 1), :] = rolled[:, : _HW]


def _select_kernel(cost_ref, offx_ref, offy_ref, ox_ref, oy_ref, corr_ref,
                   code_ref, v_ref):
    p = jax.lax.broadcasted_iota(jnp.int32, (_NUM, _HW), 1)
    y = p // _W
    x = p % _W
    dye = jnp.minimum(offy_ref[0], (_H - 1) - y)
    dxe = jnp.minimum(offx_ref[0], (_W - 1) - x)
    code = dye * 9 + dxe
    code_ref[...] = code

    def sel_body(s, u):
        plane = cost_ref[0, pl.ds(s, 1), :]
        return jnp.where(code == s, plane, u)

    u = jax.lax.fori_loop(0, _NPLANE, sel_body,
                          jnp.zeros((_NUM, _HW), jnp.float32))
    u = u / np.float32(_TEMPERATURE)

    m = jnp.max(u, axis=1, keepdims=True)
    e = jnp.exp(u - m)
    s = jnp.sum(e, axis=1, keepdims=True)
    v_ref[...] = e / s

    i_iota = jax.lax.broadcasted_iota(jnp.int32, (_NUM, _HW), 0)
    v = v_ref[...]

    def rank_body(j, rank):
        vj = v_ref[pl.ds(j, 1), :]
        beats = (vj > v) | ((vj == v) & (j < i_iota))
        return rank + beats.astype(jnp.int32)

    rank = jax.lax.fori_loop(0, _NUM, rank_body,
                             jnp.zeros((_NUM, _HW), jnp.int32))

    codef = code_ref[...]

    def out_body(k, _):
        selm = rank == k
        corr_k = jnp.sum(jnp.where(selm, v, 0.0), axis=0, keepdims=True)
        code_k = jnp.sum(jnp.where(selm, codef, 0), axis=0, keepdims=True)
        ox_ref[0, pl.ds(k, 1), :] = (code_k % 9).astype(jnp.float32)
        oy_ref[0, pl.ds(k, 1), :] = (code_k // 9).astype(jnp.float32)
        corr_ref[0, pl.ds(k, 1), :] = corr_k
        return 0

    jax.lax.fori_loop(0, _KK, out_body, 0)


@jax.jit
def kernel(left_features, right_features, offset_x, offset_y):
    offx = offset_x.reshape(_B, _NUM, _HW)
    offy = offset_y.reshape(_B, _NUM, _HW)

    cost_all = pl.pallas_call(
        _cost_kernel,
        grid=(_B, 9),
        in_specs=[
            pl.BlockSpec((1, _C, _HW), lambda b, dx: (b, 0, 0)),
            pl.BlockSpec((_C, _HW), lambda b, dx: (0, b)),
        ],
        out_specs=pl.BlockSpec((1, _PLANE_PAD, _HW), lambda b, dx: (b, 0, 0)),
        out_shape=jax.ShapeDtypeStruct((_B, _PLANE_PAD, _HW), jnp.float32),
        scratch_shapes=[
            pltpu.VMEM((_C, _LW), jnp.float32),
            pltpu.VMEM((_C, _PADW), jnp.float32),
            pltpu.VMEM((_C, _PADW), jnp.float32),
            pltpu.VMEM((16, _LW), jnp.float32),
        ],
    )(left_features, right_features)

    ox_o, oy_o, corr = pl.pallas_call(
        _select_kernel,
        grid=(_B,),
        in_specs=[
            pl.BlockSpec((1, _PLANE_PAD, _HW), lambda b: (b, 0, 0)),
            pl.BlockSpec((1, _NUM, _HW), lambda b: (b, 0, 0)),
            pl.BlockSpec((1, _NUM, _HW), lambda b: (b, 0, 0)),
        ],
        out_specs=[
            pl.BlockSpec((1, _KK, _HW), lambda b: (b, 0, 0)),
            pl.BlockSpec((1, _KK, _HW), lambda b: (b, 0, 0)),
            pl.BlockSpec((1, _KK, _HW), lambda b: (b, 0, 0)),
        ],
        out_shape=[
            jax.ShapeDtypeStruct((_B, _KK, _HW), jnp.float32),
            jax.ShapeDtypeStruct((_B, _KK, _HW), jnp.float32),
            jax.ShapeDtypeStruct((_B, _KK, _HW), jnp.float32),
        ],
        scratch_shapes=[
            pltpu.VMEM((_NUM, _HW), jnp.int32),
            pltpu.VMEM((_NUM, _HW), jnp.float32),
        ],
    )(cost_all, offx, offy)

    return (ox_o.reshape(_B, _KK, _H, _W), oy_o.reshape(_B, _KK, _H, _W),
            corr)


# speedup vs baseline: 6.0645x; 1.0709x over previous
import jax
import jax.numpy as jnp
import numpy as np
from jax.experimental import pallas as pl
from jax.experimental.pallas import tpu as pltpu

_TEMPERATURE = 0.01
_B, _C, _H, _W, _NUM = 4, 256, 64, 64, 48
_HW = _H * _W
_KK = _NUM // 3
_NPLANE = 81
_PLANE_PAD = 88
_PADW = 4736


_LW = 4224
_NG = _LW // 128
_NCT = _C // 8


def _cost_kernel(l_ref, rf_ref, out_ref, lx_ref, rbase_ref, r64_ref,
                 rowbuf_ref):
    dx = pl.program_id(1)

    @pl.when(dx == 0)
    def _():
        rbase_ref[:, : _HW] = rf_ref[...]
        rbase_ref[:, _HW:] = jnp.zeros((_C, _PADW - _HW), jnp.float32)
        r64_ref[...] = pltpu.roll(rbase_ref[...], _PADW - 64, axis=1)
        lx_ref[:, : _HW] = l_ref[0]
        lx_ref[:, _HW:] = jnp.zeros((_C, _LW - _HW), jnp.float32)

    @pl.when(dx != 0)
    def _():
        lx_ref[...] = pltpu.roll(lx_ref[...], 1, axis=1)

    def g_body(g, _):
        goff = pl.multiple_of(g * 128, 128)

        accs = [[jnp.zeros((8, 128), jnp.float32) for _ in range(9)]
                for _ in range(2)]
        for ct in range(_NCT):
            lt = lx_ref[pl.ds(ct * 8, 8), pl.ds(goff, 128)]
            for dy in range(9):
                src = rbase_ref if dy % 2 == 0 else r64_ref
                off = pl.multiple_of((g + dy // 2) * 128, 128)
                rt = src[pl.ds(ct * 8, 8), pl.ds(off, 128)]
                accs[ct % 2][dy] = accs[ct % 2][dy] + lt * rt
        for dy in range(9):
            seg = jnp.sum(accs[0][dy] + accs[1][dy], axis=0,
                          keepdims=True)
            rowbuf_ref[pl.ds(dy, 1), pl.ds(goff, 128)] = seg
        return 0

    jax.lax.fori_loop(0, _NG, g_body, 0)

    amt = jax.lax.rem(_LW - dx, _LW)
    for dy in range(9):
        row = rowbuf_ref[pl.ds(dy, 1), :]
        rolled = pltpu.roll(row, amt, axis=1)
        out_ref[0, pl.ds(dy * 9 + dx, 1), :] = rolled[:, : _HW]


def _select_kernel(cost_ref, offx_ref, offy_ref, ox_ref, oy_ref, corr_ref,
                   code_ref, v_ref, u_ref):
    p = jax.lax.broadcasted_iota(jnp.int32, (_NUM, _HW), 1)
    y = p // _W
    x = p % _W
    dye = jnp.minimum(offy_ref[0], (_H - 1) - y)
    dxe = jnp.minimum(offx_ref[0], (_W - 1) - x)
    code = dye * 9 + dxe
    code_ref[...] = code

    def pb_body(pb, _):
        po = pl.multiple_of(pb * 512, 128)
        codeb = code_ref[:, pl.ds(po, 512)]

        def sel_body(s, ub):
            plane = cost_ref[0, pl.ds(s, 1), pl.ds(po, 512)]
            return jnp.where(codeb == s, plane, ub)

        ub = jax.lax.fori_loop(0, _NPLANE, sel_body,
                               jnp.zeros((_NUM, 512), jnp.float32))
        u_ref[:, pl.ds(po, 512)] = ub
        return 0

    jax.lax.fori_loop(0, _HW // 512, pb_body, 0)
    u = u_ref[...] / np.float32(_TEMPERATURE)

    m = jnp.max(u, axis=1, keepdims=True)
    e = jnp.exp(u - m)
    s = jnp.sum(e, axis=1, keepdims=True)
    v_ref[...] = e / s

    i_iota = jax.lax.broadcasted_iota(jnp.int32, (_NUM, _HW), 0)
    v = v_ref[...]

    def rank_body(j, rank):
        vj = v_ref[pl.ds(j, 1), :]
        beats = (vj > v) | ((vj == v) & (j < i_iota))
        return rank + beats.astype(jnp.int32)

    rank = jax.lax.fori_loop(0, _NUM, rank_body,
                             jnp.zeros((_NUM, _HW), jnp.int32))

    codef = code_ref[...]

    def out_body(k, _):
        selm = rank == k
        corr_k = jnp.sum(jnp.where(selm, v, 0.0), axis=0, keepdims=True)
        code_k = jnp.sum(jnp.where(selm, codef, 0), axis=0, keepdims=True)
        ox_ref[0, pl.ds(k, 1), :] = (code_k % 9).astype(jnp.float32)
        oy_ref[0, pl.ds(k, 1), :] = (code_k // 9).astype(jnp.float32)
        corr_ref[0, pl.ds(k, 1), :] = corr_k
        return 0

    jax.lax.fori_loop(0, _KK, out_body, 0)


@jax.jit
def kernel(left_features, right_features, offset_x, offset_y):
    offx = offset_x.reshape(_B, _NUM, _HW)
    offy = offset_y.reshape(_B, _NUM, _HW)

    cost_all = pl.pallas_call(
        _cost_kernel,
        grid=(_B, 9),
        in_specs=[
            pl.BlockSpec((1, _C, _HW), lambda b, dx: (b, 0, 0)),
            pl.BlockSpec((_C, _HW), lambda b, dx: (0, b)),
        ],
        out_specs=pl.BlockSpec((1, _PLANE_PAD, _HW), lambda b, dx: (b, 0, 0)),
        out_shape=jax.ShapeDtypeStruct((_B, _PLANE_PAD, _HW), jnp.float32),
        scratch_shapes=[
            pltpu.VMEM((_C, _LW), jnp.float32),
            pltpu.VMEM((_C, _PADW), jnp.float32),
            pltpu.VMEM((_C, _PADW), jnp.float32),
            pltpu.VMEM((16, _LW), jnp.float32),
        ],
    )(left_features, right_features)

    ox_o, oy_o, corr = pl.pallas_call(
        _select_kernel,
        grid=(_B,),
        in_specs=[
            pl.BlockSpec((1, _PLANE_PAD, _HW), lambda b: (b, 0, 0)),
            pl.BlockSpec((1, _NUM, _HW), lambda b: (b, 0, 0)),
            pl.BlockSpec((1, _NUM, _HW), lambda b: (b, 0, 0)),
        ],
        out_specs=[
            pl.BlockSpec((1, _KK, _HW), lambda b: (b, 0, 0)),
            pl.BlockSpec((1, _KK, _HW), lambda b: (b, 0, 0)),
            pl.BlockSpec((1, _KK, _HW), lambda b: (b, 0, 0)),
        ],
        out_shape=[
            jax.ShapeDtypeStruct((_B, _KK, _HW), jnp.float32),
            jax.ShapeDtypeStruct((_B, _KK, _HW), jnp.float32),
            jax.ShapeDtypeStruct((_B, _KK, _HW), jnp.float32),
        ],
        scratch_shapes=[
            pltpu.VMEM((_NUM, _HW), jnp.int32),
            pltpu.VMEM((_NUM, _HW), jnp.float32),
            pltpu.VMEM((_NUM, _HW), jnp.float32),
        ],
    )(cost_all, offx, offy)

    return (ox_o.reshape(_B, _KK, _H, _W), oy_o.reshape(_B, _KK, _H, _W),
            corr)


# speedup vs baseline: 6.3460x; 1.0464x over previous
import jax
import jax.numpy as jnp
import numpy as np
from jax.experimental import pallas as pl
from jax.experimental.pallas import tpu as pltpu

_TEMPERATURE = 0.01
_B, _C, _H, _W, _NUM = 4, 256, 64, 64, 48
_HW = _H * _W
_KK = _NUM // 3
_NPLANE = 81
_PLANE_PAD = 88
_PADW = 4736


_LW = 4224
_NG = _LW // 128
_NCT = _C // 8


def _cost_kernel(l_ref, rf_ref, out_ref, lx_ref, rbase_ref, r64_ref,
                 rowbuf_ref):
    dx = pl.program_id(1)

    @pl.when(dx == 0)
    def _():
        rbase_ref[:, : _HW] = rf_ref[...]
        rbase_ref[:, _HW:] = jnp.zeros((_C, _PADW - _HW), jnp.float32)
        r64_ref[...] = pltpu.roll(rbase_ref[...], _PADW - 64, axis=1)
        lx_ref[:, : _HW] = l_ref[0]
        lx_ref[:, _HW:] = jnp.zeros((_C, _LW - _HW), jnp.float32)

    @pl.when(dx != 0)
    def _():
        lx_ref[...] = pltpu.roll(lx_ref[...], 1, axis=1)

    def g_body(g, _):
        goff = pl.multiple_of(g * 128, 128)

        accs = [[jnp.zeros((8, 128), jnp.float32) for _ in range(9)]
                for _ in range(2)]
        for ct in range(_NCT):
            lt = lx_ref[pl.ds(ct * 8, 8), pl.ds(goff, 128)]
            for dy in range(9):
                src = rbase_ref if dy % 2 == 0 else r64_ref
                off = pl.multiple_of((g + dy // 2) * 128, 128)
                rt = src[pl.ds(ct * 8, 8), pl.ds(off, 128)]
                accs[ct % 2][dy] = accs[ct % 2][dy] + lt * rt
        for dy in range(9):
            seg = jnp.sum(accs[0][dy] + accs[1][dy], axis=0,
                          keepdims=True)
            rowbuf_ref[pl.ds(dy, 1), pl.ds(goff, 128)] = seg
        return 0

    jax.lax.fori_loop(0, _NG, g_body, 0)

    amt = jax.lax.rem(_LW - dx, _LW)
    for dy in range(9):
        row = rowbuf_ref[pl.ds(dy, 1), :]
        rolled = pltpu.roll(row, amt, axis=1)
        out_ref[0, pl.ds(dy * 9 + dx, 1), :] = rolled[:, : _HW]


def _select_kernel(cost_ref, offx_ref, offy_ref, ox_ref, oy_ref, corr_ref,
                   code_ref, v_ref, u_ref, rank_ref):
    p = jax.lax.broadcasted_iota(jnp.int32, (_NUM, _HW), 1)
    y = p // _W
    x = p % _W
    dye = jnp.minimum(offy_ref[0], (_H - 1) - y)
    dxe = jnp.minimum(offx_ref[0], (_W - 1) - x)
    code = dye * 9 + dxe
    code_ref[...] = code

    def pb_body(pb, _):
        po = pl.multiple_of(pb * 512, 128)
        codeb = code_ref[:, pl.ds(po, 512)]

        def sel_body(s, ub):
            plane = cost_ref[0, pl.ds(s, 1), pl.ds(po, 512)]
            return jnp.where(codeb == s, plane, ub)

        ub = jax.lax.fori_loop(0, _NPLANE, sel_body,
                               jnp.zeros((_NUM, 512), jnp.float32))
        u_ref[:, pl.ds(po, 512)] = ub
        return 0

    jax.lax.fori_loop(0, _HW // 512, pb_body, 0)
    u = u_ref[...] / np.float32(_TEMPERATURE)

    m = jnp.max(u, axis=1, keepdims=True)
    e = jnp.exp(u - m)
    s = jnp.sum(e, axis=1, keepdims=True)
    v_ref[...] = e / s

    iib = jax.lax.broadcasted_iota(jnp.int32, (_NUM, 512), 0)

    def pb2_body(pb, _):
        po = pl.multiple_of(pb * 512, 128)
        vb = v_ref[:, pl.ds(po, 512)]

        def rank_body(j, rankb):
            vj = v_ref[pl.ds(j, 1), pl.ds(po, 512)]
            beats = (vj > vb) | ((vj == vb) & (j < iib))
            return rankb + beats.astype(jnp.int32)

        rankb = jax.lax.fori_loop(0, _NUM, rank_body,
                                  jnp.zeros((_NUM, 512), jnp.int32))
        rank_ref[:, pl.ds(po, 512)] = rankb
        return 0

    jax.lax.fori_loop(0, _HW // 512, pb2_body, 0)

    v = v_ref[...]
    rank = rank_ref[...]
    codef = code_ref[...]

    def out_body(k, _):
        selm = rank == k
        corr_k = jnp.sum(jnp.where(selm, v, 0.0), axis=0, keepdims=True)
        code_k = jnp.sum(jnp.where(selm, codef, 0), axis=0, keepdims=True)
        ox_ref[0, pl.ds(k, 1), :] = (code_k % 9).astype(jnp.float32)
        oy_ref[0, pl.ds(k, 1), :] = (code_k // 9).astype(jnp.float32)
        corr_ref[0, pl.ds(k, 1), :] = corr_k
        return 0

    jax.lax.fori_loop(0, _KK, out_body, 0)


@jax.jit
def kernel(left_features, right_features, offset_x, offset_y):
    offx = offset_x.reshape(_B, _NUM, _HW)
    offy = offset_y.reshape(_B, _NUM, _HW)

    cost_all = pl.pallas_call(
        _cost_kernel,
        grid=(_B, 9),
        in_specs=[
            pl.BlockSpec((1, _C, _HW), lambda b, dx: (b, 0, 0)),
            pl.BlockSpec((_C, _HW), lambda b, dx: (0, b)),
        ],
        out_specs=pl.BlockSpec((1, _PLANE_PAD, _HW), lambda b, dx: (b, 0, 0)),
        out_shape=jax.ShapeDtypeStruct((_B, _PLANE_PAD, _HW), jnp.float32),
        scratch_shapes=[
            pltpu.VMEM((_C, _LW), jnp.float32),
            pltpu.VMEM((_C, _PADW), jnp.float32),
            pltpu.VMEM((_C, _PADW), jnp.float32),
            pltpu.VMEM((16, _LW), jnp.float32),
        ],
    )(left_features, right_features)

    ox_o, oy_o, corr = pl.pallas_call(
        _select_kernel,
        grid=(_B,),
        in_specs=[
            pl.BlockSpec((1, _PLANE_PAD, _HW), lambda b: (b, 0, 0)),
            pl.BlockSpec((1, _NUM, _HW), lambda b: (b, 0, 0)),
            pl.BlockSpec((1, _NUM, _HW), lambda b: (b, 0, 0)),
        ],
        out_specs=[
            pl.BlockSpec((1, _KK, _HW), lambda b: (b, 0, 0)),
            pl.BlockSpec((1, _KK, _HW), lambda b: (b, 0, 0)),
            pl.BlockSpec((1, _KK, _HW), lambda b: (b, 0, 0)),
        ],
        out_shape=[
            jax.ShapeDtypeStruct((_B, _KK, _HW), jnp.float32),
            jax.ShapeDtypeStruct((_B, _KK, _HW), jnp.float32),
            jax.ShapeDtypeStruct((_B, _KK, _HW), jnp.float32),
        ],
        scratch_shapes=[
            pltpu.VMEM((_NUM, _HW), jnp.int32),
            pltpu.VMEM((_NUM, _HW), jnp.float32),
            pltpu.VMEM((_NUM, _HW), jnp.float32),
            pltpu.VMEM((_NUM, _HW), jnp.int32),
        ],
    )(cost_all, offx, offy)

    return (ox_o.reshape(_B, _KK, _H, _W), oy_o.reshape(_B, _KK, _H, _W),
            corr)
